# baseline (speedup 1.0000x reference)
import functools

import jax
import jax.numpy as jnp
from jax import lax
from jax.experimental import pallas as pl
from jax.experimental.pallas import tpu as pltpu
from jax.experimental.pallas import tpu_sc as plsc

EMB = 50
WIN = 5
VOCAB = 1000
HIDDEN = 128
OUT = 64
DPAD = 64

NC = 2
NS = 16
NW = NC * NS


def _make_sc_gather(n_idx: int):
    n_cores = 1
    n_workers = n_cores * NS
    b_per_w = n_idx // n_workers
    chunk = 640
    nbuf = 2
    n_ch = b_per_w // chunk
    mesh = plsc.VectorSubcoreMesh(
        core_axis_name="c", subcore_axis_name="s", num_cores=n_cores)

    @functools.partial(
        pl.kernel,
        mesh=mesh,
        compiler_params=pltpu.CompilerParams(use_tc_tiling_on_sc=False),
        out_type=jax.ShapeDtypeStruct((n_idx, DPAD), jnp.float32),
        scratch_types=[
            pltpu.VMEM((b_per_w,), jnp.int32),
            pltpu.VMEM((nbuf, chunk, DPAD), jnp.float32),
            pltpu.SemaphoreType.DMA,
            pltpu.SemaphoreType.DMA,
            pltpu.SemaphoreType.DMA,
            pltpu.SemaphoreType.DMA,
        ],
    )
    def sc_gather(table_hbm, idx_hbm, out_hbm, idx_v, rows_v,
                  g0, g1, s0, s1):
        wid = lax.axis_index("s") * n_cores + lax.axis_index("c")
        base = wid * b_per_w
        gsem = (g0, g1)
        ssem = (s0, s1)
        pltpu.sync_copy(idx_hbm.at[pl.ds(base, b_per_w)], idx_v)

        def gather(c):
            return pltpu.async_copy(
                table_hbm.at[idx_v.at[pl.ds(c * chunk, chunk)]],
                rows_v.at[c % nbuf], gsem[c % nbuf])

        def scatter(c):
            return pltpu.async_copy(
                rows_v.at[c % nbuf], out_hbm.at[pl.ds(base + c * chunk, chunk)],
                ssem[c % nbuf])

        gathers = [gather(c) for c in range(min(nbuf, n_ch))]
        scatters = []
        for c in range(n_ch):
            gathers[c].wait()
            scatters.append(scatter(c))
            if c + nbuf < n_ch:
                scatters[c].wait()
                gathers.append(gather(c + nbuf))
        for c in range(max(0, n_ch - nbuf), n_ch):
            scatters[c].wait()

    return sc_gather


def _mlp_body(e_ref, w1_ref, b1_ref, w2_ref, b2_ref, o_ref):
    h = lax.dot_general(
        e_ref[...], w1_ref[...], (((1,), (1,)), ((), ())),
        preferred_element_type=jnp.float32,
    ) + b1_ref[...]
    t = jnp.tanh(h)
    o = lax.dot_general(
        t, w2_ref[...], (((1,), (1,)), ((), ())),
        preferred_element_type=jnp.float32,
    ) + b2_ref[...]
    m = jnp.max(o, axis=1, keepdims=True)
    ex = jnp.exp(o - m)
    o_ref[...] = ex / jnp.sum(ex, axis=1, keepdims=True)


def _mlp(e_flat, w1p, b1, w2, b2, tile: int):
    batch = e_flat.shape[0]
    feat = e_flat.shape[1]
    return pl.pallas_call(
        _mlp_body,
        grid=(batch // tile,),
        in_specs=[
            pl.BlockSpec((tile, feat), lambda i: (i, 0)),
            pl.BlockSpec((HIDDEN, feat), lambda i: (0, 0)),
            pl.BlockSpec((1, HIDDEN), lambda i: (0, 0)),
            pl.BlockSpec((OUT, HIDDEN), lambda i: (0, 0)),
            pl.BlockSpec((1, OUT), lambda i: (0, 0)),
        ],
        out_specs=pl.BlockSpec((tile, OUT), lambda i: (i, 0)),
        out_shape=jax.ShapeDtypeStruct((batch, OUT), jnp.float32),
    )(e_flat, w1p, b1, w2, b2)


def kernel(x, embed_w, W1, b1, W2, b2):
    batch = x.shape[0]
    idx = x.reshape(-1).astype(jnp.int32)
    table = jnp.pad(embed_w, ((0, 0), (0, DPAD - EMB)))
    w1p = jnp.pad(
        W1.reshape(HIDDEN, WIN, EMB), ((0, 0), (0, 0), (0, DPAD - EMB))
    ).reshape(HIDDEN, WIN * DPAD)

    e = _make_sc_gather(idx.shape[0])(table, idx)
    e_flat = e.reshape(batch, WIN * DPAD)

    return _mlp(e_flat, w1p, b1.reshape(1, HIDDEN), W2, b2.reshape(1, OUT),
                tile=8192)


# speedup vs baseline: 1.0013x; 1.0013x over previous
import functools

import jax
import jax.numpy as jnp
from jax import lax
from jax.experimental import pallas as pl
from jax.experimental.pallas import tpu as pltpu
from jax.experimental.pallas import tpu_sc as plsc

EMB = 50
WIN = 5
VOCAB = 1000
HIDDEN = 128
OUT = 64
DPAD = 64

NS = 16


def _make_sc_gather(n_idx: int):
    n_cores = 1
    n_workers = n_cores * NS
    b_per_w = n_idx // n_workers
    chunk = 640
    nbuf = 2
    n_ch = b_per_w // chunk
    mesh = plsc.VectorSubcoreMesh(
        core_axis_name="c", subcore_axis_name="s", num_cores=n_cores)

    @functools.partial(
        pl.kernel,
        mesh=mesh,
        compiler_params=pltpu.CompilerParams(use_tc_tiling_on_sc=False),
        out_type=jax.ShapeDtypeStruct((n_idx, DPAD), jnp.float32),
        scratch_types=[
            pltpu.VMEM((b_per_w,), jnp.int32),
            pltpu.VMEM((nbuf, chunk, DPAD), jnp.float32),
            pltpu.SemaphoreType.DMA,
            pltpu.SemaphoreType.DMA,
            pltpu.SemaphoreType.DMA,
            pltpu.SemaphoreType.DMA,
        ],
    )
    def sc_gather(table_hbm, idx_hbm, out_hbm, idx_v, rows_v,
                  g0, g1, s0, s1):
        wid = lax.axis_index("s") * n_cores + lax.axis_index("c")
        base = wid * b_per_w
        gsem = (g0, g1)
        ssem = (s0, s1)
        pltpu.sync_copy(idx_hbm.at[pl.ds(base, b_per_w)], idx_v)

        def gather(c):
            return pltpu.async_copy(
                table_hbm.at[idx_v.at[pl.ds(c * chunk, chunk)]],
                rows_v.at[c % nbuf], gsem[c % nbuf])

        def scatter(c):
            return pltpu.async_copy(
                rows_v.at[c % nbuf], out_hbm.at[pl.ds(base + c * chunk, chunk)],
                ssem[c % nbuf])

        gathers = [gather(c) for c in range(min(nbuf, n_ch))]
        scatters = []
        for c in range(n_ch):
            gathers[c].wait()
            scatters.append(scatter(c))
            if c + nbuf < n_ch:
                scatters[c].wait()
                gathers.append(gather(c + nbuf))
        for c in range(max(0, n_ch - nbuf), n_ch):
            scatters[c].wait()

    return sc_gather


def _mlp_body(e_ref, w1_ref, b1_ref, w2_ref, b2_ref, o_ref):
    h = lax.dot_general(
        e_ref[...], w1_ref[...], (((1,), (1,)), ((), ())),
        preferred_element_type=jnp.float32,
    ) + b1_ref[...]
    t = jnp.tanh(h)
    o = lax.dot_general(
        t, w2_ref[...], (((1,), (1,)), ((), ())),
        preferred_element_type=jnp.float32,
    ) + b2_ref[...]
    m = jnp.max(o, axis=1, keepdims=True)
    ex = jnp.exp(o - m)
    o_ref[...] = ex / jnp.sum(ex, axis=1, keepdims=True)


def _mlp(e_flat, w1p, b1, w2, b2, tile: int):
    batch = e_flat.shape[0]
    feat = e_flat.shape[1]
    return pl.pallas_call(
        _mlp_body,
        grid=(batch // tile,),
        in_specs=[
            pl.BlockSpec((tile, feat), lambda i: (i, 0)),
            pl.BlockSpec((HIDDEN, feat), lambda i: (0, 0)),
            pl.BlockSpec((1, HIDDEN), lambda i: (0, 0)),
            pl.BlockSpec((OUT, HIDDEN), lambda i: (0, 0)),
            pl.BlockSpec((1, OUT), lambda i: (0, 0)),
        ],
        out_specs=pl.BlockSpec((tile, OUT), lambda i: (i, 0)),
        out_shape=jax.ShapeDtypeStruct((batch, OUT), jnp.float32),
    )(e_flat, w1p, b1, w2, b2)


def kernel(x, embed_w, W1, b1, W2, b2):
    batch = x.shape[0]
    idx = x.reshape(-1).astype(jnp.int32)
    table = jnp.pad(embed_w, ((0, 0), (0, DPAD - EMB)))
    w1p = jnp.pad(
        W1.reshape(HIDDEN, WIN, EMB), ((0, 0), (0, 0), (0, DPAD - EMB))
    ).reshape(HIDDEN, WIN * DPAD)

    e = _make_sc_gather(idx.shape[0])(table, idx)
    e_flat = e.reshape(batch, WIN * DPAD)

    return _mlp(e_flat, w1p, b1.reshape(1, HIDDEN), W2, b2.reshape(1, OUT),
                tile=4096)
